# Initial kernel scaffold; baseline (speedup 1.0000x reference)
import jax
import jax.numpy as jnp
from jax.experimental import pallas as pl


def kernel(x, lambda_B, lambda_Pi):
    raise NotImplementedError("write your pallas kernel here")


# speedup vs baseline: 6.3353x; 6.3353x over previous
import functools

import jax
import jax.numpy as jnp
from jax import lax
from jax.experimental import pallas as pl
from jax.experimental.pallas import tpu as pltpu
from jax.experimental.pallas import tpu_sc as plsc

N = 65536
C = 20
M = 256
J = 16
D = C * J

NC = 2
NS = 16
NW = NC * NS
ROWS_PER_W = N // NW
CHUNK = 128
NCHUNK = ROWS_PER_W // CHUNK
NBUF = 2


def _table_body(lb_ref, lpi_ref, post_ref, ll_ref):
    lam = lb_ref[:]
    mx = jnp.max(lam, axis=1, keepdims=True)
    e = jnp.exp(lam - mx)
    B = e / jnp.sum(e, axis=1, keepdims=True)
    lpi = lpi_ref[:]
    pmx = jnp.max(lpi, axis=0, keepdims=True)
    pe = jnp.exp(lpi - pmx)
    Pi = pe / jnp.sum(pe, axis=0, keepdims=True)
    T = B * Pi[:, None, :]
    denom = jnp.sum(T, axis=0)
    post_ref[:] = T / denom[None, :, :]
    ll_ref[:] = jnp.log(denom)


_table = pl.pallas_call(
    _table_body,
    out_shape=(
        jax.ShapeDtypeStruct((C, M, J), jnp.float32),
        jax.ShapeDtypeStruct((M, J), jnp.float32),
    ),
)


def _gather_body(ptab_hbm, lltab_hbm, idx_hbm, outp_hbm, outl_hbm,
                 idx_v, rows_v, llrows_v, semp, seml):
    wid = lax.axis_index("s") * NC + lax.axis_index("c")
    base = wid * ROWS_PER_W
    pltpu.sync_copy(idx_hbm.at[pl.ds(wid * NCHUNK, NCHUNK)], idx_v)

    gathers = [None] * NCHUNK

    def start(c):
        b = c % NBUF
        gp = pltpu.async_copy(ptab_hbm.at[idx_v.at[c]], rows_v.at[b], semp)
        gl = pltpu.async_copy(lltab_hbm.at[idx_v.at[c]], llrows_v.at[b], seml)
        gathers[c] = (gp, gl)

    start(0)
    for c in range(NCHUNK):
        if c + 1 < NCHUNK:
            start(c + 1)
        b = c % NBUF
        gp, gl = gathers[c]
        gp.wait()
        gl.wait()
        row0 = base + c * CHUNK
        pltpu.sync_copy(rows_v.at[b], outp_hbm.at[pl.ds(row0, CHUNK)])
        pltpu.sync_copy(llrows_v.at[b], outl_hbm.at[pl.ds(row0, CHUNK)])


@functools.cache
def _gather():
    return pl.kernel(
        _gather_body,
        mesh=plsc.VectorSubcoreMesh(
            core_axis_name="c", subcore_axis_name="s",
            num_cores=NC, num_subcores=NS),
        out_type=[
            jax.ShapeDtypeStruct((N, D), jnp.float32),
            jax.ShapeDtypeStruct((N, J), jnp.float32),
        ],
        scratch_types=[
            pltpu.VMEM((NCHUNK, CHUNK), jnp.int32),
            pltpu.VMEM((NBUF, CHUNK, D), jnp.float32),
            pltpu.VMEM((NBUF, CHUNK, J), jnp.float32),
            pltpu.SemaphoreType.DMA,
            pltpu.SemaphoreType.DMA,
        ],
        compiler_params=pltpu.CompilerParams(use_tc_tiling_on_sc=False),
    )


def kernel(x, lambda_B, lambda_Pi):
    post_t, ll_t = _table(lambda_B, lambda_Pi)
    ptab = jnp.transpose(post_t, (1, 0, 2)).reshape(M, D)
    idx = x.astype(jnp.int32).reshape(N // CHUNK, CHUNK)
    outp, outl = _gather()(ptab, ll_t, idx)
    return (outl, outp.reshape(N, C, J))
